# baseline (speedup 1.0000x reference)
import functools

import jax
import jax.numpy as jnp
from jax import lax
from jax.experimental import pallas as pl
from jax.experimental.pallas import tpu as pltpu
from jax.experimental.pallas import tpu_sc as plsc

N = 10000
E = 320000
H = 128
LBL = 10000
K = (E * 4) // 5

NC = 2
NS = 16
NW = NC * NS
PT = E // NW
CH = 80
NCHUNK = PT // CH

NBLK = 1000
EB = 3200

_mesh = plsc.VectorSubcoreMesh(core_axis_name="c", subcore_axis_name="s")
_f32 = jnp.float32
_i32 = jnp.int32


def _dense_pre_body(x_ref, ps_ref, W1_ref, b1_ref, sW1a_ref, sW1b_ref,
                    gW0_ref, Whh_ref, bhh_ref,
                    x1_ref, A_ref, B_ref, h0_ref, gh_ref):
    x = x_ref[...]
    ps = ps_ref[...]
    x1 = jax.nn.relu(jnp.dot(x, W1_ref[...], preferred_element_type=_f32)
                     + b1_ref[...])
    x1_ref[...] = x1
    A_ref[...] = jnp.dot(ps, sW1a_ref[...], preferred_element_type=_f32)
    B_ref[...] = jnp.dot(ps, sW1b_ref[...], preferred_element_type=_f32)
    h0_ref[...] = jnp.dot(x1, gW0_ref[...], preferred_element_type=_f32)
    gh_ref[...] = jnp.dot(ps, Whh_ref[...], preferred_element_type=_f32) \
        + bhh_ref[...]


def _dense_pre(x, ps, W1, b1, sW1a, sW1b, gW0, W_hh, b_hh):
    nb = N // NBLK
    row = pl.BlockSpec((NBLK, H), lambda i: (i, 0))
    full = lambda s: pl.BlockSpec(s, lambda i: tuple(0 for _ in s))
    return pl.pallas_call(
        _dense_pre_body,
        grid=(nb,),
        in_specs=[row, row, full((H, H)), full((1, H)), full((H, H)),
                  full((H, H)), full((H, H)), full((H, 3 * H)),
                  full((1, 3 * H))],
        out_specs=[row, row, row, row,
                   pl.BlockSpec((NBLK, 3 * H), lambda i: (i, 0))],
        out_shape=[jax.ShapeDtypeStruct((N, H), _f32)] * 4
        + [jax.ShapeDtypeStruct((N, 3 * H), _f32)],
    )(x, ps, W1, b1, sW1a, sW1b, gW0, W_hh, b_hh)


def _gather_add_body(A_hbm, B_hbm, src3_hbm, dst3_hbm, G_hbm,
                     sidx, didx, bufA0, bufA1, bufB0, bufB1,
                     semA0, semA1, semB0, semB1, wsem0, wsem1):
    wid = lax.axis_index("s") * NC + lax.axis_index("c")
    pltpu.sync_copy(src3_hbm.at[wid], sidx)
    pltpu.sync_copy(dst3_hbm.at[wid], didx)

    slots = ((bufA0, bufB0, semA0, semB0, wsem0),
             (bufA1, bufB1, semA1, semB1, wsem1))

    def issue(ch, s):
        bA, bB, sA, sB, _ = slots[s]
        pltpu.async_copy(A_hbm.at[sidx.at[ch]], bA, sA)
        pltpu.async_copy(B_hbm.at[didx.at[ch]], bB, sB)

    issue(0, 0)

    def process(i, s):
        bA, bB, sA, sB, ws = slots[s]
        bAn, bBn, _, _, wsn = slots[1 - s]

        @pl.when(i + 1 < NCHUNK)
        def _():
            @pl.when(i >= 1)
            def _():
                pltpu.make_async_copy(bAn, G_hbm.at[pl.ds(0, CH)], wsn).wait()
            issue(i + 1, 1 - s)

        pltpu.make_async_copy(A_hbm.at[pl.ds(0, CH)], bA, sA).wait()
        pltpu.make_async_copy(B_hbm.at[pl.ds(0, CH)], bB, sB).wait()

        def addrow(r, _):
            for j in range(H // 16):
                sl = pl.ds(j * 16, 16)
                bA[r, sl] = bA[r, sl] + bB[r, sl]
            return 0

        lax.fori_loop(0, CH, addrow, 0)
        pltpu.async_copy(bA, G_hbm.at[pl.ds(wid * PT + i * CH, CH)], ws)

    def body(i, _):
        @pl.when(i % 2 == 0)
        def _():
            process(i, 0)

        @pl.when(i % 2 == 1)
        def _():
            process(i, 1)
        return 0

    lax.fori_loop(0, NCHUNK, body, 0)
    pltpu.make_async_copy(bufA0, G_hbm.at[pl.ds(0, CH)], wsem0).wait()
    pltpu.make_async_copy(bufA1, G_hbm.at[pl.ds(0, CH)], wsem1).wait()


def _gather_add(A, B, src3, dst3):
    return pl.kernel(
        _gather_add_body,
        out_type=jax.ShapeDtypeStruct((E, H), _f32),
        mesh=_mesh,
        compiler_params=pltpu.CompilerParams(needs_layout_passes=False),
        scratch_types=[
            pltpu.VMEM((NCHUNK, CH), _i32),
            pltpu.VMEM((NCHUNK, CH), _i32),
            pltpu.VMEM((CH, H), _f32),
            pltpu.VMEM((CH, H), _f32),
            pltpu.VMEM((CH, H), _f32),
            pltpu.VMEM((CH, H), _f32),
            pltpu.SemaphoreType.DMA,
            pltpu.SemaphoreType.DMA,
            pltpu.SemaphoreType.DMA,
            pltpu.SemaphoreType.DMA,
            pltpu.SemaphoreType.DMA,
            pltpu.SemaphoreType.DMA,
        ],
    )(A, B, src3, dst3)


def _min_body(ef_ref, out_ref):
    out_ref[...] = jnp.min(ef_ref[...]).reshape(1, 1)


def _min_reduce(ef2):
    return pl.pallas_call(
        _min_body,
        grid=(1,),
        in_specs=[pl.BlockSpec(ef2.shape, lambda i: (0, 0))],
        out_specs=pl.BlockSpec((1, 1), lambda i: (0, 0)),
        out_shape=jax.ShapeDtypeStruct((1, 1), _f32),
    )(ef2)


ND = 24


def _logits_body(G_ref, ef_ref, m_ref, R_ref, sb1_ref, sW2_ref, sb2_ref,
                 lg_ref):
    u = ef_ref[0] - (m_ref[0, 0] + 0.5)
    plist = [jnp.ones((1, EB), _f32)]
    for _ in range(ND - 1):
        plist.append(plist[-1] * u)
    V = jnp.concatenate(plist, axis=0)
    C = lax.dot_general(V, R_ref[...], (((0,), (0,)), ((), ())),
                        preferred_element_type=_f32)
    hid = jax.nn.relu(G_ref[...] + C + sb1_ref[...])
    lrow = lax.dot_general(sW2_ref[...], hid, (((0,), (1,)), ((), ())),
                           preferred_element_type=_f32)
    lg_ref[...] = (lrow + sb2_ref[0, 0])[None]


def _logits(G, ef3, m, R, sb1, sW2, sb2):
    nb = E // EB
    full = lambda s: pl.BlockSpec(s, lambda i: tuple(0 for _ in s))
    return pl.pallas_call(
        _logits_body,
        grid=(nb,),
        in_specs=[pl.BlockSpec((EB, H), lambda i: (i, 0)),
                  pl.BlockSpec((1, 1, EB), lambda i: (i, 0, 0)),
                  full((1, 1)), full((ND, H)), full((1, H)), full((H, 1)),
                  full((1, 1))],
        out_specs=pl.BlockSpec((1, 1, EB), lambda i: (i, 0, 0)),
        out_shape=jax.ShapeDtypeStruct((nb, 1, EB), _f32),
    )(G, ef3, m, R, sb1, sW2, sb2)


def _threshold_body(lg_ref, w_ref):
    z = jax.nn.sigmoid(lg_ref[...])
    bits = lax.bitcast_convert_type(z, _i32)
    rows, cols = z.shape
    ridx = lax.broadcasted_iota(_i32, (rows, cols), 0)
    cidx = lax.broadcasted_iota(_i32, (rows, cols), 1)
    idx = ridx * cols + cidx

    kf = jnp.float32(K)

    def cnt_ge(v):
        return jnp.sum((bits >= v).astype(_f32))

    def bit_step(i, tau):
        cand = tau | lax.shift_left(jnp.int32(1), 30 - i)
        return lax.cond(cnt_ge(cand) >= kf, lambda: cand, lambda: tau)

    tau = lax.fori_loop(0, 31, bit_step, jnp.int32(0))

    g_cnt = jnp.sum((bits > tau).astype(_f32))
    t_need = kf - g_cnt
    tie = bits == tau

    def tie_cnt(c):
        return jnp.sum((tie & (idx < c)).astype(_f32))

    def idx_step(i, c):
        cand = c | lax.shift_left(jnp.int32(1), 19 - i)
        return lax.cond(tie_cnt(cand) <= t_need, lambda: cand, lambda: c)

    cutoff = lax.fori_loop(0, 20, idx_step, jnp.int32(0))

    keep = (bits > tau) | (tie & (idx < cutoff))
    w_ref[...] = jnp.where(keep, z, 0.0)


def _threshold(lg2):
    return pl.pallas_call(
        _threshold_body,
        grid=(1,),
        in_specs=[pl.BlockSpec(lg2.shape, lambda i: (0, 0))],
        out_specs=pl.BlockSpec(lg2.shape, lambda i: (0, 0)),
        out_shape=jax.ShapeDtypeStruct(lg2.shape, _f32),
    )(lg2)


_DR = N // 16


def _deg_body(dst_hbm, w_hbm, degp_hbm, dstl, wl, degl, sem0):
    wid = lax.axis_index("s") * NC + lax.axis_index("c")
    pltpu.sync_copy(dst_hbm.at[pl.ds(wid * PT, PT)], dstl)
    pltpu.sync_copy(w_hbm.at[pl.ds(wid * PT, PT)], wl)

    def zrow(i, _):
        degl[pl.ds(i * 16, 16)] = jnp.zeros((16,), _f32)
        return 0

    lax.fori_loop(0, NP_ // 16, zrow, 0)

    def step(i, _):
        sl = pl.ds(i * 16, 16)
        d = dstl[sl]
        ww = wl[sl]
        plsc.addupdate_scatter(degl, [d], ww)
        return 0

    lax.fori_loop(0, PT // 16, step, 0)
    pltpu.sync_copy(degl, degp_hbm.at[pl.ds(wid * NP_, NP_)])


def _deg_scatter(dst, w):
    return pl.kernel(
        _deg_body,
        out_type=jax.ShapeDtypeStruct((NW * NP_,), _f32),
        mesh=_mesh,
        scratch_types=[
            pltpu.VMEM((PT,), _i32),
            pltpu.VMEM((PT,), _f32),
            pltpu.VMEM((NP_,), _f32),
            pltpu.SemaphoreType.DMA,
        ],
        compiler_params=pltpu.CompilerParams(needs_layout_passes=False),
    )(dst, w)


NP_ = 10240
DBLK = 1024


def _dinv_body(degp_ref, dinv_ref):
    deg_row = 1.0 + jnp.sum(degp_ref[...], axis=0, keepdims=True)
    dinv_ref[...] = jnp.transpose(lax.rsqrt(deg_row))


def _dinv_kernel(degp2):
    nb = NP_ // DBLK
    return pl.pallas_call(
        _dinv_body,
        grid=(nb,),
        in_specs=[pl.BlockSpec((NW, DBLK), lambda i: (0, i))],
        out_specs=pl.BlockSpec((DBLK, 1), lambda i: (i, 0)),
        out_shape=jax.ShapeDtypeStruct((NP_, 1), _f32),
    )(degp2)


N_PAD = 10240
_STRIPE = N_PAD // NS
CHG = 32
E_PAD = NW * 10240
PTG = E_PAD // NW
NCHUNK_G = PTG // CHG


def _gcn_body(hp_hbm, dinv_hbm, src_hbm, dst_hbm, w_hbm, accp_hbm,
              si0, si1, di0, di1, dsc0, dsc1, wb0, wb1, bufR0, bufR1,
              sb0, sb1, dtbl, wd, acc_sh,
              isem0, isem1, gsem0, gsem1, ssem0, ssem1):
    cid = lax.axis_index("c")
    sid = lax.axis_index("s")
    wid = sid * NC + cid

    pltpu.sync_copy(dinv_hbm, dtbl)

    def zrow(r, _):
        for j in range(H // 16):
            bufR0[r, pl.ds(j * 16, 16)] = jnp.zeros((16,), _f32)
        return 0

    lax.fori_loop(0, CHG, zrow, 0)
    s0 = sid * _STRIPE
    for q in range(_STRIPE // CHG):
        pltpu.sync_copy(bufR0, acc_sh.at[pl.ds(s0 + q * CHG, CHG)])
    plsc.subcore_barrier()

    slots = ((si0, di0, dsc0, wb0, bufR0, sb0, isem0, gsem0, ssem0),
             (si1, di1, dsc1, wb1, bufR1, sb1, isem1, gsem1, ssem1))

    def issue_idx(ch, s):
        si, di, _, wb, _, _, isem, _, _ = slots[s]
        base = wid * PTG + ch * CHG
        pltpu.async_copy(src_hbm.at[pl.ds(base, CHG)], si, isem)
        pltpu.async_copy(dst_hbm.at[pl.ds(base, CHG)], di, isem)
        pltpu.async_copy(w_hbm.at[pl.ds(base, CHG)], wb, isem)

    def wait_idx(s):
        si, di, _, wb, _, _, isem, _, _ = slots[s]
        pltpu.make_async_copy(src_hbm.at[pl.ds(0, CHG)], si, isem).wait()
        pltpu.make_async_copy(dst_hbm.at[pl.ds(0, CHG)], di, isem).wait()
        pltpu.make_async_copy(w_hbm.at[pl.ds(0, CHG)], wb, isem).wait()

    def issue_gather(s):
        si, _, _, _, bR, _, _, gsem, _ = slots[s]
        hh = CHG // 2
        pltpu.async_copy(hp_hbm.at[si.at[pl.ds(0, hh)]],
                         bR.at[pl.ds(0, hh)], gsem)
        pltpu.async_copy(hp_hbm.at[si.at[pl.ds(hh, hh)]],
                         bR.at[pl.ds(hh, hh)], gsem)

    def drain_scat(s):
        _, _, _, _, _, sb, _, _, ssem = slots[s]
        pltpu.make_async_copy(sb, acc_sh.at[pl.ds(0, CHG)], ssem).wait()

    issue_idx(0, 0)
    wait_idx(0)
    issue_gather(0)
    issue_idx(1, 1)

    def process(i, s):
        si, di, dsc, wb, bR, sb, isem, gsem, ssem = slots[s]

        @pl.when(i + 1 < NCHUNK_G)
        def _():
            wait_idx(1 - s)
            issue_gather(1 - s)

        def mkwd(g, _):
            sl = pl.ds(g * 16, 16)
            wd[sl] = wb[sl] * plsc.load_gather(dtbl, [si[sl]])
            dsc[sl] = di[sl]
            return 0

        lax.fori_loop(0, CHG // 16, mkwd, 0)

        pltpu.make_async_copy(hp_hbm.at[pl.ds(0, CHG)], bR, gsem).wait()

        @pl.when(i >= 2)
        def _():
            drain_scat(s)

        def scale(r, _):
            wv = plsc.load_gather(wd, [jnp.full((16,), r, _i32)])
            for j in range(H // 16):
                sl = pl.ds(j * 16, 16)
                sb[r, sl] = bR[r, sl] * wv
            return 0

        lax.fori_loop(0, CHG, scale, 0)
        pltpu.async_copy(sb, acc_sh.at[dsc], ssem, add=True)

        @pl.when(i + 2 < NCHUNK_G)
        def _():
            issue_idx(i + 2, s)

    def body(i, _):
        @pl.when(i % 2 == 0)
        def _():
            process(i, 0)

        @pl.when(i % 2 == 1)
        def _():
            process(i, 1)
        return 0

    lax.fori_loop(0, NCHUNK_G, body, 0)
    drain_scat(0)
    drain_scat(1)
    plsc.subcore_barrier()

    for q in range(_STRIPE // CHG):
        pltpu.sync_copy(acc_sh.at[pl.ds(s0 + q * CHG, CHG)],
                        accp_hbm.at[cid, pl.ds(s0 + q * CHG, CHG)])


def _gcn_scatter(hp, dinv_flat, srcp, dstp, wp):
    return pl.kernel(
        _gcn_body,
        out_type=jax.ShapeDtypeStruct((NC, N_PAD, H), _f32),
        mesh=_mesh,
        compiler_params=pltpu.CompilerParams(needs_layout_passes=False),
        scratch_types=[
            pltpu.VMEM((CHG,), _i32),
            pltpu.VMEM((CHG,), _i32),
            pltpu.VMEM((CHG,), _i32),
            pltpu.VMEM((CHG,), _i32),
            pltpu.VMEM((CHG,), _i32),
            pltpu.VMEM((CHG,), _i32),
            pltpu.VMEM((CHG,), _f32),
            pltpu.VMEM((CHG,), _f32),
            pltpu.VMEM((CHG, H), _f32),
            pltpu.VMEM((CHG, H), _f32),
            pltpu.VMEM((CHG, H), _f32),
            pltpu.VMEM((CHG, H), _f32),
            pltpu.VMEM((NP_,), _f32),
            pltpu.VMEM((CHG,), _f32),
            pltpu.VMEM_SHARED((N_PAD, H), _f32),
            pltpu.SemaphoreType.DMA,
            pltpu.SemaphoreType.DMA,
            pltpu.SemaphoreType.DMA,
            pltpu.SemaphoreType.DMA,
            pltpu.SemaphoreType.DMA,
            pltpu.SemaphoreType.DMA,
        ],
    )(hp, dinv_flat, srcp, dstp, wp)


def _gcn0_fin_body(accp_ref, h0_ref, dinv_ref, gb0_ref, gW1_ref, h1_ref):
    di = dinv_ref[...]
    acc = accp_ref[0] + accp_ref[1]
    x2 = jax.nn.relu(di * acc + di * di * h0_ref[...] + gb0_ref[...])
    h1_ref[...] = jnp.dot(x2, gW1_ref[...], preferred_element_type=_f32)


def _gcn0_finish(accp, h0, dinv, gb0, gW1):
    nb = N // NBLK
    full = lambda s: pl.BlockSpec(s, lambda i: tuple(0 for _ in s))
    return pl.pallas_call(
        _gcn0_fin_body,
        grid=(nb,),
        in_specs=[pl.BlockSpec((NC, NBLK, H), lambda i: (0, i, 0)),
                  pl.BlockSpec((NBLK, H), lambda i: (i, 0)),
                  pl.BlockSpec((NBLK, 1), lambda i: (i, 0)),
                  full((1, H)), full((H, H))],
        out_specs=pl.BlockSpec((NBLK, H), lambda i: (i, 0)),
        out_shape=jax.ShapeDtypeStruct((N, H), _f32),
    )(accp, h0, dinv, gb0, gW1)


def _gru_body(accp_ref, h1_ref, dinv_ref, gb1_ref, gh_ref, ps_ref,
              Wih_ref, bih_ref, hnew_ref):
    di = dinv_ref[...]
    acc = accp_ref[0] + accp_ref[1]
    x3 = di * acc + di * di * h1_ref[...] + gb1_ref[...]
    gi = jnp.dot(x3, Wih_ref[...], preferred_element_type=_f32) + bih_ref[...]
    gh = gh_ref[...]
    r = jax.nn.sigmoid(gi[:, 0:H] + gh[:, 0:H])
    zg = jax.nn.sigmoid(gi[:, H:2 * H] + gh[:, H:2 * H])
    n = jnp.tanh(gi[:, 2 * H:3 * H] + r * gh[:, 2 * H:3 * H])
    hnew_ref[...] = (1.0 - zg) * n + zg * ps_ref[...]


def _gcn1_gru(accp, h1, dinv, gb1, gh, ps, W_ih, b_ih):
    nb = N // NBLK
    full = lambda s: pl.BlockSpec(s, lambda i: tuple(0 for _ in s))
    return pl.pallas_call(
        _gru_body,
        grid=(nb,),
        in_specs=[pl.BlockSpec((NC, NBLK, H), lambda i: (0, i, 0)),
                  pl.BlockSpec((NBLK, H), lambda i: (i, 0)),
                  pl.BlockSpec((NBLK, 1), lambda i: (i, 0)),
                  full((1, H)),
                  pl.BlockSpec((NBLK, 3 * H), lambda i: (i, 0)),
                  pl.BlockSpec((NBLK, H), lambda i: (i, 0)),
                  full((H, 3 * H)), full((1, 3 * H))],
        out_specs=pl.BlockSpec((NBLK, H), lambda i: (i, 0)),
        out_shape=jax.ShapeDtypeStruct((N, H), _f32),
    )(accp, h1, dinv, gb1, gh, ps, W_ih, b_ih)


_LPAD = 20480
_LPT = _LPAD // NW
_LCH = _LPT // CH


def _dec_gather_body(hn_hbm, lidx3_hbm, out_hbm, idxb, buf0, buf1,
                     sem0, sem1, wsem0, wsem1):
    wid = lax.axis_index("s") * NC + lax.axis_index("c")
    pltpu.sync_copy(lidx3_hbm.at[wid], idxb)

    slots = ((buf0, sem0, wsem0), (buf1, sem1, wsem1))

    def issue(ch, s):
        b, sm, _ = slots[s]
        pltpu.async_copy(hn_hbm.at[idxb.at[ch]], b, sm)

    issue(0, 0)

    def process(i, s):
        b, sm, ws = slots[s]
        bn, _, wsn = slots[1 - s]

        @pl.when(i + 1 < _LCH)
        def _():
            @pl.when(i >= 1)
            def _():
                pltpu.make_async_copy(bn, out_hbm.at[pl.ds(0, CH)], wsn).wait()
            issue(i + 1, 1 - s)

        pltpu.make_async_copy(hn_hbm.at[pl.ds(0, CH)], b, sm).wait()
        pltpu.async_copy(b, out_hbm.at[pl.ds(wid * _LPT + i * CH, CH)], ws)

    def body(i, _):
        @pl.when(i % 2 == 0)
        def _():
            process(i, 0)

        @pl.when(i % 2 == 1)
        def _():
            process(i, 1)
        return 0

    lax.fori_loop(0, _LCH, body, 0)
    pltpu.make_async_copy(buf0, out_hbm.at[pl.ds(0, CH)], wsem0).wait()
    pltpu.make_async_copy(buf1, out_hbm.at[pl.ds(0, CH)], wsem1).wait()


def _dec_gather(h_new, lidx3):
    return pl.kernel(
        _dec_gather_body,
        out_type=jax.ShapeDtypeStruct((_LPAD, H), _f32),
        mesh=_mesh,
        compiler_params=pltpu.CompilerParams(needs_layout_passes=False),
        scratch_types=[
            pltpu.VMEM((_LCH, CH), _i32),
            pltpu.VMEM((CH, H), _f32),
            pltpu.VMEM((CH, H), _f32),
            pltpu.SemaphoreType.DMA,
            pltpu.SemaphoreType.DMA,
            pltpu.SemaphoreType.DMA,
            pltpu.SemaphoreType.DMA,
        ],
    )(h_new, lidx3)


def _dec_body(Hs_ref, Hd_ref, dW1a_ref, dW1b_ref, db1_ref, dW2_ref, db2_ref,
              pred_ref):
    hid = jax.nn.relu(
        jnp.dot(Hs_ref[...], dW1a_ref[...], preferred_element_type=_f32)
        + jnp.dot(Hd_ref[...], dW1b_ref[...], preferred_element_type=_f32)
        + db1_ref[...])
    pred_ref[...] = jnp.dot(hid, dW2_ref[...], preferred_element_type=_f32) \
        + db2_ref[0, 0]


def _decoder(Hs, Hd, dW1a, dW1b, db1, dW2, db2):
    nb = LBL // NBLK
    full = lambda s: pl.BlockSpec(s, lambda i: tuple(0 for _ in s))
    return pl.pallas_call(
        _dec_body,
        grid=(nb,),
        in_specs=[pl.BlockSpec((NBLK, H), lambda i: (i, 0)),
                  pl.BlockSpec((NBLK, H), lambda i: (i, 0)),
                  full((H, H)), full((H, H)), full((1, H)), full((H, 1)),
                  full((1, 1))],
        out_specs=pl.BlockSpec((NBLK, 1), lambda i: (i, 0)),
        out_shape=jax.ShapeDtypeStruct((LBL, 1), _f32),
    )(Hs, Hd, dW1a, dW1b, db1, dW2, db2)


def kernel(x, edge_index, edge_label_index, edge_feature, previous_state,
           W1, b1, time_w, time_b, sW1, sb1, sW2, sb2, gW0, gb0, gW1, gb1,
           W_ih, W_hh, b_ih, b_hh, dW1, db1, dW2, db2):
    src = edge_index[0]
    dst = edge_index[1]
    sW1a = sW1[0:H]
    sW1b = sW1[H:2 * H]
    sW1c = sW1[2 * H:3 * H]
    b1r = b1.reshape(1, H)
    sb1r = sb1.reshape(1, H)
    sb2r = sb2.reshape(1, 1)
    tbr = time_b.reshape(1, H)
    gb0r = gb0.reshape(1, H)
    gb1r = gb1.reshape(1, H)
    bihr = b_ih.reshape(1, 3 * H)
    bhhr = b_hh.reshape(1, 3 * H)
    db1r = db1.reshape(1, H)
    db2r = db2.reshape(1, 1)

    src3 = src.reshape(NW, NCHUNK, CH)
    dst3 = dst.reshape(NW, NCHUNK, CH)

    x1, A, B, h0, gh = _dense_pre(x, previous_state, W1, b1r, sW1a, sW1b,
                                  gW0, W_hh, bhhr)

    G = _gather_add(A, B, src3, dst3)

    m = _min_reduce(edge_feature.reshape(E // H, H))

    wv = time_w[0]
    bv = time_b
    theta = 1.5 * wv + bv
    dd = jnp.arange(ND, dtype=_f32)
    ratio = jnp.where(dd[:, None] > 0, wv[None, :] / jnp.maximum(dd[:, None], 1.0),
                      jnp.ones((1, H), _f32))
    M = jnp.cumprod(ratio, axis=0)
    kk = jnp.floor_divide(jnp.arange(ND), 2).astype(_f32)
    sgn = jnp.where(jnp.mod(kk, 2.0) == 0.0, 1.0, -1.0)
    even = jnp.mod(jnp.arange(ND), 2) == 0
    gam = jnp.where(even[:, None], sgn[:, None] * jnp.cos(theta)[None, :],
                    -sgn[:, None] * jnp.sin(theta)[None, :])
    R = (M * gam) @ sW1c

    lg = _logits(G, edge_feature.reshape(E // EB, 1, EB), m, R, sb1r,
                 sW2, sb2r)

    w2 = _threshold(lg.reshape(E // H, H))
    w = w2.reshape(E)

    degp = _deg_scatter(dst, w)
    dinv = _dinv_kernel(degp.reshape(NW, NP_))
    dinv_flat = dinv.reshape(NP_)

    pad = jnp.zeros((E_PAD - E,), _i32)
    srcp = jnp.concatenate([src, pad])
    dstp = jnp.concatenate([dst, pad])
    wp = jnp.concatenate([w, jnp.zeros((E_PAD - E,), _f32)])
    accp0 = _gcn_scatter(h0, dinv_flat, srcp, dstp, wp)
    h1 = _gcn0_finish(accp0, h0, dinv, gb0r, gW1)

    accp1 = _gcn_scatter(h1, dinv_flat, srcp, dstp, wp)
    h_new = _gcn1_gru(accp1, h1, dinv, gb1r, gh, previous_state, W_ih, bihr)

    lidx = jnp.concatenate([edge_label_index[0], edge_label_index[1]])
    lidx_pad = jnp.concatenate(
        [lidx, jnp.zeros((_LPAD - 2 * LBL,), _i32)])
    HG = _dec_gather(h_new, lidx_pad.reshape(NW, _LCH, CH))
    Hs = HG[0:LBL]
    Hd = HG[LBL:2 * LBL]

    dW1a = dW1[0:H]
    dW1b = dW1[H:2 * H]
    pred = _decoder(Hs, Hd, dW1a, dW1b, db1r, dW2, db2r)
    return (pred, h_new)


# speedup vs baseline: 1.2251x; 1.2251x over previous
import functools

import jax
import jax.numpy as jnp
from jax import lax
from jax.experimental import pallas as pl
from jax.experimental.pallas import tpu as pltpu
from jax.experimental.pallas import tpu_sc as plsc

N = 10000
E = 320000
H = 128
LBL = 10000
K = (E * 4) // 5

NC = 2
NS = 16
NW = NC * NS
PT = E // NW
CH = 80
NCHUNK = PT // CH

NBLK = 1000
EB = 3200

_mesh = plsc.VectorSubcoreMesh(core_axis_name="c", subcore_axis_name="s")
_f32 = jnp.float32
_i32 = jnp.int32


def _dense_pre_body(x_ref, ps_ref, W1_ref, b1_ref, sW1a_ref, sW1b_ref,
                    gW0_ref, Whh_ref, bhh_ref,
                    x1_ref, A_ref, B_ref, h0_ref, gh_ref):
    x = x_ref[...]
    ps = ps_ref[...]
    x1 = jax.nn.relu(jnp.dot(x, W1_ref[...], preferred_element_type=_f32)
                     + b1_ref[...])
    x1_ref[...] = x1
    A_ref[...] = jnp.dot(ps, sW1a_ref[...], preferred_element_type=_f32)
    B_ref[...] = jnp.dot(ps, sW1b_ref[...], preferred_element_type=_f32)
    h0_ref[...] = jnp.dot(x1, gW0_ref[...], preferred_element_type=_f32)
    gh_ref[...] = jnp.dot(ps, Whh_ref[...], preferred_element_type=_f32) \
        + bhh_ref[...]


def _dense_pre(x, ps, W1, b1, sW1a, sW1b, gW0, W_hh, b_hh):
    nb = N // NBLK
    row = pl.BlockSpec((NBLK, H), lambda i: (i, 0))
    full = lambda s: pl.BlockSpec(s, lambda i: tuple(0 for _ in s))
    return pl.pallas_call(
        _dense_pre_body,
        grid=(nb,),
        in_specs=[row, row, full((H, H)), full((1, H)), full((H, H)),
                  full((H, H)), full((H, H)), full((H, 3 * H)),
                  full((1, 3 * H))],
        out_specs=[row, row, row, row,
                   pl.BlockSpec((NBLK, 3 * H), lambda i: (i, 0))],
        out_shape=[jax.ShapeDtypeStruct((N, H), _f32)] * 4
        + [jax.ShapeDtypeStruct((N, 3 * H), _f32)],
    )(x, ps, W1, b1, sW1a, sW1b, gW0, W_hh, b_hh)


def _gather_add_body(A_hbm, B_hbm, src3_hbm, dst3_hbm, G_hbm,
                     sidx, didx, bufA0, bufA1, bufB0, bufB1,
                     semA0, semA1, semB0, semB1, wsem0, wsem1):
    wid = lax.axis_index("s") * NC + lax.axis_index("c")
    pltpu.sync_copy(src3_hbm.at[wid], sidx)
    pltpu.sync_copy(dst3_hbm.at[wid], didx)

    slots = ((bufA0, bufB0, semA0, semB0, wsem0),
             (bufA1, bufB1, semA1, semB1, wsem1))

    def issue(ch, s):
        bA, bB, sA, sB, _ = slots[s]
        pltpu.async_copy(A_hbm.at[sidx.at[ch]], bA, sA)
        pltpu.async_copy(B_hbm.at[didx.at[ch]], bB, sB)

    issue(0, 0)

    def process(i, s):
        bA, bB, sA, sB, ws = slots[s]
        bAn, bBn, _, _, wsn = slots[1 - s]

        @pl.when(i + 1 < NCHUNK)
        def _():
            @pl.when(i >= 1)
            def _():
                pltpu.make_async_copy(bAn, G_hbm.at[pl.ds(0, CH)], wsn).wait()
            issue(i + 1, 1 - s)

        pltpu.make_async_copy(A_hbm.at[pl.ds(0, CH)], bA, sA).wait()
        pltpu.make_async_copy(B_hbm.at[pl.ds(0, CH)], bB, sB).wait()

        def addrow(r, _):
            for j in range(H // 16):
                sl = pl.ds(j * 16, 16)
                bA[r, sl] = bA[r, sl] + bB[r, sl]
            return 0

        lax.fori_loop(0, CH, addrow, 0)
        pltpu.async_copy(bA, G_hbm.at[pl.ds(wid * PT + i * CH, CH)], ws)

    def body(i, _):
        @pl.when(i % 2 == 0)
        def _():
            process(i, 0)

        @pl.when(i % 2 == 1)
        def _():
            process(i, 1)
        return 0

    lax.fori_loop(0, NCHUNK, body, 0)
    pltpu.make_async_copy(bufA0, G_hbm.at[pl.ds(0, CH)], wsem0).wait()
    pltpu.make_async_copy(bufA1, G_hbm.at[pl.ds(0, CH)], wsem1).wait()


def _gather_add(A, B, src3, dst3):
    return pl.kernel(
        _gather_add_body,
        out_type=jax.ShapeDtypeStruct((E, H), _f32),
        mesh=_mesh,
        compiler_params=pltpu.CompilerParams(needs_layout_passes=False),
        scratch_types=[
            pltpu.VMEM((NCHUNK, CH), _i32),
            pltpu.VMEM((NCHUNK, CH), _i32),
            pltpu.VMEM((CH, H), _f32),
            pltpu.VMEM((CH, H), _f32),
            pltpu.VMEM((CH, H), _f32),
            pltpu.VMEM((CH, H), _f32),
            pltpu.SemaphoreType.DMA,
            pltpu.SemaphoreType.DMA,
            pltpu.SemaphoreType.DMA,
            pltpu.SemaphoreType.DMA,
            pltpu.SemaphoreType.DMA,
            pltpu.SemaphoreType.DMA,
        ],
    )(A, B, src3, dst3)


def _min_body(ef_ref, out_ref):
    out_ref[...] = jnp.min(ef_ref[...]).reshape(1, 1)


def _min_reduce(ef2):
    return pl.pallas_call(
        _min_body,
        grid=(1,),
        in_specs=[pl.BlockSpec(ef2.shape, lambda i: (0, 0))],
        out_specs=pl.BlockSpec((1, 1), lambda i: (0, 0)),
        out_shape=jax.ShapeDtypeStruct((1, 1), _f32),
    )(ef2)


ND = 24


def _logits_body(G_ref, ef_ref, m_ref, R_ref, sb1_ref, sW2_ref, sb2_ref,
                 lg_ref):
    u = ef_ref[0] - (m_ref[0, 0] + 0.5)
    plist = [jnp.ones((1, EB), _f32)]
    for _ in range(ND - 1):
        plist.append(plist[-1] * u)
    V = jnp.concatenate(plist, axis=0)
    C = lax.dot_general(V, R_ref[...], (((0,), (0,)), ((), ())),
                        preferred_element_type=_f32)
    hid = jax.nn.relu(G_ref[...] + C + sb1_ref[...])
    lrow = lax.dot_general(sW2_ref[...], hid, (((0,), (1,)), ((), ())),
                           preferred_element_type=_f32)
    lg_ref[...] = (lrow + sb2_ref[0, 0])[None]


def _logits(G, ef3, m, R, sb1, sW2, sb2):
    nb = E // EB
    full = lambda s: pl.BlockSpec(s, lambda i: tuple(0 for _ in s))
    return pl.pallas_call(
        _logits_body,
        grid=(nb,),
        in_specs=[pl.BlockSpec((EB, H), lambda i: (i, 0)),
                  pl.BlockSpec((1, 1, EB), lambda i: (i, 0, 0)),
                  full((1, 1)), full((ND, H)), full((1, H)), full((H, 1)),
                  full((1, 1))],
        out_specs=pl.BlockSpec((1, 1, EB), lambda i: (i, 0, 0)),
        out_shape=jax.ShapeDtypeStruct((nb, 1, EB), _f32),
    )(G, ef3, m, R, sb1, sW2, sb2)


def _threshold_body(lg_ref, w_ref):
    z = jax.nn.sigmoid(lg_ref[...])
    bits = lax.bitcast_convert_type(z, _i32)
    rows, cols = z.shape
    ridx = lax.broadcasted_iota(_i32, (rows, cols), 0)
    cidx = lax.broadcasted_iota(_i32, (rows, cols), 1)
    idx = ridx * cols + cidx

    kf = jnp.float32(K)

    def cnt_ge(v):
        return jnp.sum((bits >= v).astype(_f32))

    def bit_step(i, tau):
        cand = tau | lax.shift_left(jnp.int32(1), 30 - i)
        return lax.cond(cnt_ge(cand) >= kf, lambda: cand, lambda: tau)

    tau = lax.fori_loop(0, 31, bit_step, jnp.int32(0))

    g_cnt = jnp.sum((bits > tau).astype(_f32))
    t_need = kf - g_cnt
    tie = bits == tau

    def tie_cnt(c):
        return jnp.sum((tie & (idx < c)).astype(_f32))

    def idx_step(i, c):
        cand = c | lax.shift_left(jnp.int32(1), 19 - i)
        return lax.cond(tie_cnt(cand) <= t_need, lambda: cand, lambda: c)

    cutoff = lax.fori_loop(0, 20, idx_step, jnp.int32(0))

    keep = (bits > tau) | (tie & (idx < cutoff))
    w_ref[...] = jnp.where(keep, z, 0.0)


def _threshold(lg2):
    return pl.pallas_call(
        _threshold_body,
        grid=(1,),
        in_specs=[pl.BlockSpec(lg2.shape, lambda i: (0, 0))],
        out_specs=pl.BlockSpec(lg2.shape, lambda i: (0, 0)),
        out_shape=jax.ShapeDtypeStruct(lg2.shape, _f32),
    )(lg2)


_DR = N // 16


def _deg_body(dst_hbm, w_hbm, degp_hbm, dstl, wl, degl, sem0):
    wid = lax.axis_index("s") * NC + lax.axis_index("c")
    pltpu.sync_copy(dst_hbm.at[pl.ds(wid * PT, PT)], dstl)
    pltpu.sync_copy(w_hbm.at[pl.ds(wid * PT, PT)], wl)

    def zrow(i, _):
        degl[pl.ds(i * 16, 16)] = jnp.zeros((16,), _f32)
        return 0

    lax.fori_loop(0, NP_ // 16, zrow, 0)

    def step(i, _):
        sl = pl.ds(i * 16, 16)
        d = dstl[sl]
        ww = wl[sl]
        plsc.addupdate_scatter(degl, [d], ww)
        return 0

    lax.fori_loop(0, PT // 16, step, 0)
    pltpu.sync_copy(degl, degp_hbm.at[pl.ds(wid * NP_, NP_)])


def _deg_scatter(dst, w):
    return pl.kernel(
        _deg_body,
        out_type=jax.ShapeDtypeStruct((NW * NP_,), _f32),
        mesh=_mesh,
        scratch_types=[
            pltpu.VMEM((PT,), _i32),
            pltpu.VMEM((PT,), _f32),
            pltpu.VMEM((NP_,), _f32),
            pltpu.SemaphoreType.DMA,
        ],
        compiler_params=pltpu.CompilerParams(needs_layout_passes=False),
    )(dst, w)


NP_ = 10240
DBLK = 1024


def _dinv_body(degp_ref, dinv_ref):
    deg_row = 1.0 + jnp.sum(degp_ref[...], axis=0, keepdims=True)
    dinv_ref[...] = jnp.transpose(lax.rsqrt(deg_row))


def _dinv_kernel(degp2):
    nb = NP_ // DBLK
    return pl.pallas_call(
        _dinv_body,
        grid=(nb,),
        in_specs=[pl.BlockSpec((NW, DBLK), lambda i: (0, i))],
        out_specs=pl.BlockSpec((DBLK, 1), lambda i: (i, 0)),
        out_shape=jax.ShapeDtypeStruct((NP_, 1), _f32),
    )(degp2)


N_PAD = 10240
_STRIPE = N_PAD // NS
CHG = 64
E_PAD = NW * 10240
PTG = E_PAD // NW
NCHUNK_G = PTG // CHG


def _gcn_body(hp_hbm, dinv_hbm, src_hbm, dst_hbm, w_hbm, accp_hbm,
              si0, si1, di0, di1, dsc0, dsc1, wb0, wb1, bufR0, bufR1,
              dtbl, wd, acc_sh, isem0, isem1, gsem0, gsem1, ssem0, ssem1):
    cid = lax.axis_index("c")
    sid = lax.axis_index("s")
    wid = sid * NC + cid

    pltpu.sync_copy(dinv_hbm, dtbl)

    def zrow(r, _):
        for j in range(H // 16):
            bufR0[r, pl.ds(j * 16, 16)] = jnp.zeros((16,), _f32)
        return 0

    lax.fori_loop(0, CHG, zrow, 0)
    s0 = sid * _STRIPE
    for q in range(_STRIPE // CHG):
        pltpu.sync_copy(bufR0, acc_sh.at[pl.ds(s0 + q * CHG, CHG)])
    plsc.subcore_barrier()

    slots = ((si0, di0, dsc0, wb0, bufR0, isem0, gsem0, ssem0),
             (si1, di1, dsc1, wb1, bufR1, isem1, gsem1, ssem1))

    def issue_idx(ch, s):
        si, di, _, wb, _, isem, _, _ = slots[s]
        base = wid * PTG + ch * CHG
        pltpu.async_copy(src_hbm.at[pl.ds(base, CHG)], si, isem)
        pltpu.async_copy(dst_hbm.at[pl.ds(base, CHG)], di, isem)
        pltpu.async_copy(w_hbm.at[pl.ds(base, CHG)], wb, isem)

    def wait_idx(s):
        si, di, _, wb, _, isem, _, _ = slots[s]
        pltpu.make_async_copy(src_hbm.at[pl.ds(0, CHG)], si, isem).wait()
        pltpu.make_async_copy(dst_hbm.at[pl.ds(0, CHG)], di, isem).wait()
        pltpu.make_async_copy(w_hbm.at[pl.ds(0, CHG)], wb, isem).wait()

    def issue_gather(s):
        si, _, _, _, bR, _, gsem, _ = slots[s]
        pltpu.async_copy(hp_hbm.at[si], bR, gsem)

    def drain_scat(s):
        _, _, dsc_, _, bR, _, _, ssem = slots[s]
        pltpu.make_async_copy(bR, acc_sh.at[pl.ds(0, CHG)], ssem).wait()

    issue_idx(0, 0)
    wait_idx(0)
    issue_gather(0)
    issue_idx(1, 1)

    def process(i, s):
        si, di, dsc, wb, bR, isem, gsem, ssem = slots[s]

        @pl.when(i + 1 < NCHUNK_G)
        def _():
            @pl.when(i >= 1)
            def _():
                drain_scat(1 - s)
            wait_idx(1 - s)
            issue_gather(1 - s)

        def mkwd(g, _):
            sl = pl.ds(g * 16, 16)
            wd[sl] = wb[sl] * plsc.load_gather(dtbl, [si[sl]])
            dsc[sl] = di[sl]
            return 0

        lax.fori_loop(0, CHG // 16, mkwd, 0)

        pltpu.make_async_copy(hp_hbm.at[pl.ds(0, CHG)], bR, gsem).wait()

        def scale(r, _):
            wv = plsc.load_gather(wd, [jnp.full((16,), r, _i32)])
            for j in range(H // 16):
                sl = pl.ds(j * 16, 16)
                bR[r, sl] = bR[r, sl] * wv
            return 0

        lax.fori_loop(0, CHG, scale, 0)
        pltpu.async_copy(bR, acc_sh.at[dsc], ssem, add=True)

        @pl.when(i + 2 < NCHUNK_G)
        def _():
            issue_idx(i + 2, s)

    def body(i, _):
        @pl.when(i % 2 == 0)
        def _():
            process(i, 0)

        @pl.when(i % 2 == 1)
        def _():
            process(i, 1)
        return 0

    lax.fori_loop(0, NCHUNK_G, body, 0)
    drain_scat(0)
    drain_scat(1)
    plsc.subcore_barrier()

    for q in range(_STRIPE // CHG):
        pltpu.sync_copy(acc_sh.at[pl.ds(s0 + q * CHG, CHG)],
                        accp_hbm.at[cid, pl.ds(s0 + q * CHG, CHG)])


def _gcn_scatter(hp, dinv_flat, srcp, dstp, wp):
    return pl.kernel(
        _gcn_body,
        out_type=jax.ShapeDtypeStruct((NC, N_PAD, H), _f32),
        mesh=_mesh,
        compiler_params=pltpu.CompilerParams(needs_layout_passes=False),
        scratch_types=[
            pltpu.VMEM((CHG,), _i32),
            pltpu.VMEM((CHG,), _i32),
            pltpu.VMEM((CHG,), _i32),
            pltpu.VMEM((CHG,), _i32),
            pltpu.VMEM((CHG,), _i32),
            pltpu.VMEM((CHG,), _i32),
            pltpu.VMEM((CHG,), _f32),
            pltpu.VMEM((CHG,), _f32),
            pltpu.VMEM((CHG, H), _f32),
            pltpu.VMEM((CHG, H), _f32),
            pltpu.VMEM((NP_,), _f32),
            pltpu.VMEM((CHG,), _f32),
            pltpu.VMEM_SHARED((N_PAD, H), _f32),
            pltpu.SemaphoreType.DMA,
            pltpu.SemaphoreType.DMA,
            pltpu.SemaphoreType.DMA,
            pltpu.SemaphoreType.DMA,
            pltpu.SemaphoreType.DMA,
            pltpu.SemaphoreType.DMA,
        ],
    )(hp, dinv_flat, srcp, dstp, wp)


def _gcn0_fin_body(accp_ref, h0_ref, dinv_ref, gb0_ref, gW1_ref, h1_ref):
    di = dinv_ref[...]
    acc = accp_ref[0] + accp_ref[1]
    x2 = jax.nn.relu(di * acc + di * di * h0_ref[...] + gb0_ref[...])
    h1_ref[...] = jnp.dot(x2, gW1_ref[...], preferred_element_type=_f32)


def _gcn0_finish(accp, h0, dinv, gb0, gW1):
    nb = N // NBLK
    full = lambda s: pl.BlockSpec(s, lambda i: tuple(0 for _ in s))
    return pl.pallas_call(
        _gcn0_fin_body,
        grid=(nb,),
        in_specs=[pl.BlockSpec((NC, NBLK, H), lambda i: (0, i, 0)),
                  pl.BlockSpec((NBLK, H), lambda i: (i, 0)),
                  pl.BlockSpec((NBLK, 1), lambda i: (i, 0)),
                  full((1, H)), full((H, H))],
        out_specs=pl.BlockSpec((NBLK, H), lambda i: (i, 0)),
        out_shape=jax.ShapeDtypeStruct((N, H), _f32),
    )(accp, h0, dinv, gb0, gW1)


def _gru_body(accp_ref, h1_ref, dinv_ref, gb1_ref, gh_ref, ps_ref,
              Wih_ref, bih_ref, hnew_ref):
    di = dinv_ref[...]
    acc = accp_ref[0] + accp_ref[1]
    x3 = di * acc + di * di * h1_ref[...] + gb1_ref[...]
    gi = jnp.dot(x3, Wih_ref[...], preferred_element_type=_f32) + bih_ref[...]
    gh = gh_ref[...]
    r = jax.nn.sigmoid(gi[:, 0:H] + gh[:, 0:H])
    zg = jax.nn.sigmoid(gi[:, H:2 * H] + gh[:, H:2 * H])
    n = jnp.tanh(gi[:, 2 * H:3 * H] + r * gh[:, 2 * H:3 * H])
    hnew_ref[...] = (1.0 - zg) * n + zg * ps_ref[...]


def _gcn1_gru(accp, h1, dinv, gb1, gh, ps, W_ih, b_ih):
    nb = N // NBLK
    full = lambda s: pl.BlockSpec(s, lambda i: tuple(0 for _ in s))
    return pl.pallas_call(
        _gru_body,
        grid=(nb,),
        in_specs=[pl.BlockSpec((NC, NBLK, H), lambda i: (0, i, 0)),
                  pl.BlockSpec((NBLK, H), lambda i: (i, 0)),
                  pl.BlockSpec((NBLK, 1), lambda i: (i, 0)),
                  full((1, H)),
                  pl.BlockSpec((NBLK, 3 * H), lambda i: (i, 0)),
                  pl.BlockSpec((NBLK, H), lambda i: (i, 0)),
                  full((H, 3 * H)), full((1, 3 * H))],
        out_specs=pl.BlockSpec((NBLK, H), lambda i: (i, 0)),
        out_shape=jax.ShapeDtypeStruct((N, H), _f32),
    )(accp, h1, dinv, gb1, gh, ps, W_ih, b_ih)


_LPAD = 20480
_LPT = _LPAD // NW
_LCH = _LPT // CH


def _dec_gather_body(hn_hbm, lidx3_hbm, out_hbm, idxb, buf0, buf1,
                     sem0, sem1, wsem0, wsem1):
    wid = lax.axis_index("s") * NC + lax.axis_index("c")
    pltpu.sync_copy(lidx3_hbm.at[wid], idxb)

    slots = ((buf0, sem0, wsem0), (buf1, sem1, wsem1))

    def issue(ch, s):
        b, sm, _ = slots[s]
        pltpu.async_copy(hn_hbm.at[idxb.at[ch]], b, sm)

    issue(0, 0)

    def process(i, s):
        b, sm, ws = slots[s]
        bn, _, wsn = slots[1 - s]

        @pl.when(i + 1 < _LCH)
        def _():
            @pl.when(i >= 1)
            def _():
                pltpu.make_async_copy(bn, out_hbm.at[pl.ds(0, CH)], wsn).wait()
            issue(i + 1, 1 - s)

        pltpu.make_async_copy(hn_hbm.at[pl.ds(0, CH)], b, sm).wait()
        pltpu.async_copy(b, out_hbm.at[pl.ds(wid * _LPT + i * CH, CH)], ws)

    def body(i, _):
        @pl.when(i % 2 == 0)
        def _():
            process(i, 0)

        @pl.when(i % 2 == 1)
        def _():
            process(i, 1)
        return 0

    lax.fori_loop(0, _LCH, body, 0)
    pltpu.make_async_copy(buf0, out_hbm.at[pl.ds(0, CH)], wsem0).wait()
    pltpu.make_async_copy(buf1, out_hbm.at[pl.ds(0, CH)], wsem1).wait()


def _dec_gather(h_new, lidx3):
    return pl.kernel(
        _dec_gather_body,
        out_type=jax.ShapeDtypeStruct((_LPAD, H), _f32),
        mesh=_mesh,
        compiler_params=pltpu.CompilerParams(needs_layout_passes=False),
        scratch_types=[
            pltpu.VMEM((_LCH, CH), _i32),
            pltpu.VMEM((CH, H), _f32),
            pltpu.VMEM((CH, H), _f32),
            pltpu.SemaphoreType.DMA,
            pltpu.SemaphoreType.DMA,
            pltpu.SemaphoreType.DMA,
            pltpu.SemaphoreType.DMA,
        ],
    )(h_new, lidx3)


def _dec_body(Hs_ref, Hd_ref, dW1a_ref, dW1b_ref, db1_ref, dW2_ref, db2_ref,
              pred_ref):
    hid = jax.nn.relu(
        jnp.dot(Hs_ref[...], dW1a_ref[...], preferred_element_type=_f32)
        + jnp.dot(Hd_ref[...], dW1b_ref[...], preferred_element_type=_f32)
        + db1_ref[...])
    pred_ref[...] = jnp.dot(hid, dW2_ref[...], preferred_element_type=_f32) \
        + db2_ref[0, 0]


def _decoder(Hs, Hd, dW1a, dW1b, db1, dW2, db2):
    nb = LBL // NBLK
    full = lambda s: pl.BlockSpec(s, lambda i: tuple(0 for _ in s))
    return pl.pallas_call(
        _dec_body,
        grid=(nb,),
        in_specs=[pl.BlockSpec((NBLK, H), lambda i: (i, 0)),
                  pl.BlockSpec((NBLK, H), lambda i: (i, 0)),
                  full((H, H)), full((H, H)), full((1, H)), full((H, 1)),
                  full((1, 1))],
        out_specs=pl.BlockSpec((NBLK, 1), lambda i: (i, 0)),
        out_shape=jax.ShapeDtypeStruct((LBL, 1), _f32),
    )(Hs, Hd, dW1a, dW1b, db1, dW2, db2)


def kernel(x, edge_index, edge_label_index, edge_feature, previous_state,
           W1, b1, time_w, time_b, sW1, sb1, sW2, sb2, gW0, gb0, gW1, gb1,
           W_ih, W_hh, b_ih, b_hh, dW1, db1, dW2, db2):
    src = edge_index[0]
    dst = edge_index[1]
    sW1a = sW1[0:H]
    sW1b = sW1[H:2 * H]
    sW1c = sW1[2 * H:3 * H]
    b1r = b1.reshape(1, H)
    sb1r = sb1.reshape(1, H)
    sb2r = sb2.reshape(1, 1)
    tbr = time_b.reshape(1, H)
    gb0r = gb0.reshape(1, H)
    gb1r = gb1.reshape(1, H)
    bihr = b_ih.reshape(1, 3 * H)
    bhhr = b_hh.reshape(1, 3 * H)
    db1r = db1.reshape(1, H)
    db2r = db2.reshape(1, 1)

    src3 = src.reshape(NW, NCHUNK, CH)
    dst3 = dst.reshape(NW, NCHUNK, CH)

    x1, A, B, h0, gh = _dense_pre(x, previous_state, W1, b1r, sW1a, sW1b,
                                  gW0, W_hh, bhhr)

    G = _gather_add(A, B, src3, dst3)

    m = _min_reduce(edge_feature.reshape(E // H, H))

    wv = time_w[0]
    bv = time_b
    theta = 1.5 * wv + bv
    dd = jnp.arange(ND, dtype=_f32)
    ratio = jnp.where(dd[:, None] > 0, wv[None, :] / jnp.maximum(dd[:, None], 1.0),
                      jnp.ones((1, H), _f32))
    M = jnp.cumprod(ratio, axis=0)
    kk = jnp.floor_divide(jnp.arange(ND), 2).astype(_f32)
    sgn = jnp.where(jnp.mod(kk, 2.0) == 0.0, 1.0, -1.0)
    even = jnp.mod(jnp.arange(ND), 2) == 0
    gam = jnp.where(even[:, None], sgn[:, None] * jnp.cos(theta)[None, :],
                    -sgn[:, None] * jnp.sin(theta)[None, :])
    R = (M * gam) @ sW1c

    lg = _logits(G, edge_feature.reshape(E // EB, 1, EB), m, R, sb1r,
                 sW2, sb2r)

    w2 = _threshold(lg.reshape(E // H, H))
    w = w2.reshape(E)

    degp = _deg_scatter(dst, w)
    dinv = _dinv_kernel(degp.reshape(NW, NP_))
    dinv_flat = dinv.reshape(NP_)

    pad = jnp.zeros((E_PAD - E,), _i32)
    srcp = jnp.concatenate([src, pad])
    dstp = jnp.concatenate([dst, pad])
    wp = jnp.concatenate([w, jnp.zeros((E_PAD - E,), _f32)])
    accp0 = _gcn_scatter(h0, dinv_flat, srcp, dstp, wp)
    h1 = _gcn0_finish(accp0, h0, dinv, gb0r, gW1)

    accp1 = _gcn_scatter(h1, dinv_flat, srcp, dstp, wp)
    h_new = _gcn1_gru(accp1, h1, dinv, gb1r, gh, previous_state, W_ih, bihr)

    lidx = jnp.concatenate([edge_label_index[0], edge_label_index[1]])
    lidx_pad = jnp.concatenate(
        [lidx, jnp.zeros((_LPAD - 2 * LBL,), _i32)])
    HG = _dec_gather(h_new, lidx_pad.reshape(NW, _LCH, CH))
    Hs = HG[0:LBL]
    Hd = HG[LBL:2 * LBL]

    dW1a = dW1[0:H]
    dW1b = dW1[H:2 * H]
    pred = _decoder(Hs, Hd, dW1a, dW1b, db1r, dW2, db2r)
    return (pred, h_new)


# speedup vs baseline: 1.2862x; 1.0499x over previous
import functools

import jax
import jax.numpy as jnp
from jax import lax
from jax.experimental import pallas as pl
from jax.experimental.pallas import tpu as pltpu
from jax.experimental.pallas import tpu_sc as plsc

N = 10000
E = 320000
H = 128
LBL = 10000
K = (E * 4) // 5

NC = 2
NS = 16
NW = NC * NS
PT = E // NW
CH = 80
NCHUNK = PT // CH

NBLK = 1000
EB = 3200

_mesh = plsc.VectorSubcoreMesh(core_axis_name="c", subcore_axis_name="s")
_f32 = jnp.float32
_i32 = jnp.int32


def _dense_pre_body(x_ref, ps_ref, W1_ref, b1_ref, sW1a_ref, sW1b_ref,
                    gW0_ref, Whh_ref, bhh_ref,
                    x1_ref, A_ref, B_ref, h0_ref, gh_ref):
    x = x_ref[...]
    ps = ps_ref[...]
    x1 = jax.nn.relu(jnp.dot(x, W1_ref[...], preferred_element_type=_f32)
                     + b1_ref[...])
    x1_ref[...] = x1
    A_ref[...] = jnp.dot(ps, sW1a_ref[...], preferred_element_type=_f32)
    B_ref[...] = jnp.dot(ps, sW1b_ref[...], preferred_element_type=_f32)
    h0_ref[...] = jnp.dot(x1, gW0_ref[...], preferred_element_type=_f32)
    gh_ref[...] = jnp.dot(ps, Whh_ref[...], preferred_element_type=_f32) \
        + bhh_ref[...]


def _dense_pre(x, ps, W1, b1, sW1a, sW1b, gW0, W_hh, b_hh):
    nb = N // NBLK
    row = pl.BlockSpec((NBLK, H), lambda i: (i, 0))
    full = lambda s: pl.BlockSpec(s, lambda i: tuple(0 for _ in s))
    return pl.pallas_call(
        _dense_pre_body,
        grid=(nb,),
        in_specs=[row, row, full((H, H)), full((1, H)), full((H, H)),
                  full((H, H)), full((H, H)), full((H, 3 * H)),
                  full((1, 3 * H))],
        out_specs=[row, row, row, row,
                   pl.BlockSpec((NBLK, 3 * H), lambda i: (i, 0))],
        out_shape=[jax.ShapeDtypeStruct((N, H), _f32)] * 4
        + [jax.ShapeDtypeStruct((N, 3 * H), _f32)],
    )(x, ps, W1, b1, sW1a, sW1b, gW0, W_hh, b_hh)


def _gather_add_body(A_hbm, B_hbm, src3_hbm, dst3_hbm, G_hbm,
                     sidx, didx, bufA0, bufA1, bufB0, bufB1,
                     semA0, semA1, semB0, semB1, wsem0, wsem1):
    wid = lax.axis_index("s") * NC + lax.axis_index("c")
    pltpu.sync_copy(src3_hbm.at[wid], sidx)
    pltpu.sync_copy(dst3_hbm.at[wid], didx)

    slots = ((bufA0, bufB0, semA0, semB0, wsem0),
             (bufA1, bufB1, semA1, semB1, wsem1))

    def issue(ch, s):
        bA, bB, sA, sB, _ = slots[s]
        pltpu.async_copy(A_hbm.at[sidx.at[ch]], bA, sA)
        pltpu.async_copy(B_hbm.at[didx.at[ch]], bB, sB)

    issue(0, 0)

    def process(i, s):
        bA, bB, sA, sB, ws = slots[s]
        bAn, bBn, _, _, wsn = slots[1 - s]

        @pl.when(i + 1 < NCHUNK)
        def _():
            @pl.when(i >= 1)
            def _():
                pltpu.make_async_copy(bAn, G_hbm.at[pl.ds(0, CH)], wsn).wait()
            issue(i + 1, 1 - s)

        pltpu.make_async_copy(A_hbm.at[pl.ds(0, CH)], bA, sA).wait()
        pltpu.make_async_copy(B_hbm.at[pl.ds(0, CH)], bB, sB).wait()

        def addrow(r, _):
            for j in range(H // 16):
                sl = pl.ds(j * 16, 16)
                bA[r, sl] = bA[r, sl] + bB[r, sl]
            return 0

        lax.fori_loop(0, CH, addrow, 0)
        pltpu.async_copy(bA, G_hbm.at[pl.ds(wid * PT + i * CH, CH)], ws)

    def body(i, _):
        @pl.when(i % 2 == 0)
        def _():
            process(i, 0)

        @pl.when(i % 2 == 1)
        def _():
            process(i, 1)
        return 0

    lax.fori_loop(0, NCHUNK, body, 0)
    pltpu.make_async_copy(bufA0, G_hbm.at[pl.ds(0, CH)], wsem0).wait()
    pltpu.make_async_copy(bufA1, G_hbm.at[pl.ds(0, CH)], wsem1).wait()


def _gather_add(A, B, src3, dst3):
    return pl.kernel(
        _gather_add_body,
        out_type=jax.ShapeDtypeStruct((E, H), _f32),
        mesh=_mesh,
        compiler_params=pltpu.CompilerParams(needs_layout_passes=False),
        scratch_types=[
            pltpu.VMEM((NCHUNK, CH), _i32),
            pltpu.VMEM((NCHUNK, CH), _i32),
            pltpu.VMEM((CH, H), _f32),
            pltpu.VMEM((CH, H), _f32),
            pltpu.VMEM((CH, H), _f32),
            pltpu.VMEM((CH, H), _f32),
            pltpu.SemaphoreType.DMA,
            pltpu.SemaphoreType.DMA,
            pltpu.SemaphoreType.DMA,
            pltpu.SemaphoreType.DMA,
            pltpu.SemaphoreType.DMA,
            pltpu.SemaphoreType.DMA,
        ],
    )(A, B, src3, dst3)


def _min_body(ef_ref, out_ref):
    out_ref[...] = jnp.min(ef_ref[...]).reshape(1, 1)


def _min_reduce(ef2):
    return pl.pallas_call(
        _min_body,
        grid=(1,),
        in_specs=[pl.BlockSpec(ef2.shape, lambda i: (0, 0))],
        out_specs=pl.BlockSpec((1, 1), lambda i: (0, 0)),
        out_shape=jax.ShapeDtypeStruct((1, 1), _f32),
    )(ef2)


ND = 24


def _logits_body(G_ref, ef_ref, m_ref, R_ref, sb1_ref, sW2_ref, sb2_ref,
                 lg_ref):
    u = ef_ref[0] - (m_ref[0, 0] + 0.5)
    plist = [jnp.ones((1, EB), _f32)]
    for _ in range(ND - 1):
        plist.append(plist[-1] * u)
    V = jnp.concatenate(plist, axis=0)
    C = lax.dot_general(V, R_ref[...], (((0,), (0,)), ((), ())),
                        preferred_element_type=_f32)
    hid = jax.nn.relu(G_ref[...] + C + sb1_ref[...])
    lrow = lax.dot_general(sW2_ref[...], hid, (((0,), (1,)), ((), ())),
                           preferred_element_type=_f32)
    lg_ref[...] = (lrow + sb2_ref[0, 0])[None]


def _logits(G, ef3, m, R, sb1, sW2, sb2):
    nb = E // EB
    full = lambda s: pl.BlockSpec(s, lambda i: tuple(0 for _ in s))
    return pl.pallas_call(
        _logits_body,
        grid=(nb,),
        in_specs=[pl.BlockSpec((EB, H), lambda i: (i, 0)),
                  pl.BlockSpec((1, 1, EB), lambda i: (i, 0, 0)),
                  full((1, 1)), full((ND, H)), full((1, H)), full((H, 1)),
                  full((1, 1))],
        out_specs=pl.BlockSpec((1, 1, EB), lambda i: (i, 0, 0)),
        out_shape=jax.ShapeDtypeStruct((nb, 1, EB), _f32),
    )(G, ef3, m, R, sb1, sW2, sb2)


def _threshold_body(lg_ref, w_ref):
    z = jax.nn.sigmoid(lg_ref[...])
    bits = lax.bitcast_convert_type(z, _i32)
    rows, cols = z.shape
    ridx = lax.broadcasted_iota(_i32, (rows, cols), 0)
    cidx = lax.broadcasted_iota(_i32, (rows, cols), 1)
    idx = ridx * cols + cidx

    kf = jnp.float32(K)

    def cnt_ge(v):
        return jnp.sum((bits >= v).astype(_f32))

    def bit_step(i, tau):
        cand = tau | lax.shift_left(jnp.int32(1), 30 - i)
        return lax.cond(cnt_ge(cand) >= kf, lambda: cand, lambda: tau)

    tau = lax.fori_loop(0, 31, bit_step, jnp.int32(0))

    g_cnt = jnp.sum((bits > tau).astype(_f32))
    t_need = kf - g_cnt
    tie = bits == tau

    def tie_cnt(c):
        return jnp.sum((tie & (idx < c)).astype(_f32))

    def idx_step(i, c):
        cand = c | lax.shift_left(jnp.int32(1), 19 - i)
        return lax.cond(tie_cnt(cand) <= t_need, lambda: cand, lambda: c)

    cutoff = lax.fori_loop(0, 20, idx_step, jnp.int32(0))

    keep = (bits > tau) | (tie & (idx < cutoff))
    w_ref[...] = jnp.where(keep, z, 0.0)


def _threshold(lg2):
    return pl.pallas_call(
        _threshold_body,
        grid=(1,),
        in_specs=[pl.BlockSpec(lg2.shape, lambda i: (0, 0))],
        out_specs=pl.BlockSpec(lg2.shape, lambda i: (0, 0)),
        out_shape=jax.ShapeDtypeStruct(lg2.shape, _f32),
    )(lg2)


_DR = N // 16


def _deg_body(dst_hbm, w_hbm, degp_hbm, dstl, wl, degl, sem0):
    wid = lax.axis_index("s") * NC + lax.axis_index("c")
    pltpu.sync_copy(dst_hbm.at[pl.ds(wid * PT, PT)], dstl)
    pltpu.sync_copy(w_hbm.at[pl.ds(wid * PT, PT)], wl)

    def zrow(i, _):
        degl[pl.ds(i * 16, 16)] = jnp.zeros((16,), _f32)
        return 0

    lax.fori_loop(0, NP_ // 16, zrow, 0)

    def step(i, _):
        sl = pl.ds(i * 16, 16)
        d = dstl[sl]
        ww = wl[sl]
        plsc.addupdate_scatter(degl, [d], ww)
        return 0

    lax.fori_loop(0, PT // 16, step, 0)
    pltpu.sync_copy(degl, degp_hbm.at[pl.ds(wid * NP_, NP_)])


def _deg_scatter(dst, w):
    return pl.kernel(
        _deg_body,
        out_type=jax.ShapeDtypeStruct((NW * NP_,), _f32),
        mesh=_mesh,
        scratch_types=[
            pltpu.VMEM((PT,), _i32),
            pltpu.VMEM((PT,), _f32),
            pltpu.VMEM((NP_,), _f32),
            pltpu.SemaphoreType.DMA,
        ],
        compiler_params=pltpu.CompilerParams(needs_layout_passes=False),
    )(dst, w)


NP_ = 10240
DBLK = 1024


def _dinv_body(degp_ref, dinv_ref):
    deg_row = 1.0 + jnp.sum(degp_ref[...], axis=0, keepdims=True)
    dinv_ref[...] = jnp.transpose(lax.rsqrt(deg_row))


def _dinv_kernel(degp2):
    nb = NP_ // DBLK
    return pl.pallas_call(
        _dinv_body,
        grid=(nb,),
        in_specs=[pl.BlockSpec((NW, DBLK), lambda i: (0, i))],
        out_specs=pl.BlockSpec((DBLK, 1), lambda i: (i, 0)),
        out_shape=jax.ShapeDtypeStruct((NP_, 1), _f32),
    )(degp2)


N_PAD = 10240
_STRIPE = N_PAD // NS
CHG = 64
E_PAD = NW * 10240
PAIR = 2 * 10240
C0E = 12544
NCH0 = C0E // CHG
NCH1 = (PAIR - C0E) // CHG
NCHUNK_G = NCH0


def _gcn_body(hp_hbm, dinv_hbm, src_hbm, dst_hbm, w_hbm, accp_hbm,
              si0, si1, di0, di1, dsc0, dsc1, wb0, wb1, bufR0, bufR1,
              dtbl, wd, acc_sh, isem0, isem1, gsem0, gsem1, ssem0, ssem1):
    cid = lax.axis_index("c")
    sid = lax.axis_index("s")
    wid = sid * NC + cid

    pltpu.sync_copy(dinv_hbm, dtbl)

    def zrow(r, _):
        for j in range(H // 16):
            bufR0[r, pl.ds(j * 16, 16)] = jnp.zeros((16,), _f32)
        return 0

    lax.fori_loop(0, CHG, zrow, 0)
    s0 = sid * _STRIPE
    for q in range(_STRIPE // CHG):
        pltpu.sync_copy(bufR0, acc_sh.at[pl.ds(s0 + q * CHG, CHG)])
    plsc.subcore_barrier()

    slots = ((si0, di0, dsc0, wb0, bufR0, isem0, gsem0, ssem0),
             (si1, di1, dsc1, wb1, bufR1, isem1, gsem1, ssem1))

    nchunk = jnp.where(cid == 0, NCH0, NCH1)
    tbase = sid * PAIR + cid * C0E

    def issue_idx(ch, s):
        si, di, _, wb, _, isem, _, _ = slots[s]
        base = tbase + ch * CHG
        pltpu.async_copy(src_hbm.at[pl.ds(base, CHG)], si, isem)
        pltpu.async_copy(dst_hbm.at[pl.ds(base, CHG)], di, isem)
        pltpu.async_copy(w_hbm.at[pl.ds(base, CHG)], wb, isem)

    def wait_idx(s):
        si, di, _, wb, _, isem, _, _ = slots[s]
        pltpu.make_async_copy(src_hbm.at[pl.ds(0, CHG)], si, isem).wait()
        pltpu.make_async_copy(dst_hbm.at[pl.ds(0, CHG)], di, isem).wait()
        pltpu.make_async_copy(w_hbm.at[pl.ds(0, CHG)], wb, isem).wait()

    def issue_gather(s):
        si, _, _, _, bR, _, gsem, _ = slots[s]
        pltpu.async_copy(hp_hbm.at[si], bR, gsem)

    def drain_scat(s):
        _, _, dsc_, _, bR, _, _, ssem = slots[s]
        pltpu.make_async_copy(bR, acc_sh.at[pl.ds(0, CHG)], ssem).wait()

    issue_idx(0, 0)
    wait_idx(0)
    issue_gather(0)
    issue_idx(1, 1)

    def process(i, s):
        si, di, dsc, wb, bR, isem, gsem, ssem = slots[s]

        @pl.when(i + 1 < nchunk)
        def _():
            @pl.when(i >= 1)
            def _():
                drain_scat(1 - s)
            wait_idx(1 - s)
            issue_gather(1 - s)

        def mkwd(g, _):
            sl = pl.ds(g * 16, 16)
            wd[sl] = wb[sl] * plsc.load_gather(dtbl, [si[sl]])
            dsc[sl] = di[sl]
            return 0

        lax.fori_loop(0, CHG // 16, mkwd, 0)

        pltpu.make_async_copy(hp_hbm.at[pl.ds(0, CHG)], bR, gsem).wait()

        def scale(r, _):
            wv = plsc.load_gather(wd, [jnp.full((16,), r, _i32)])
            for j in range(H // 16):
                sl = pl.ds(j * 16, 16)
                bR[r, sl] = bR[r, sl] * wv
            return 0

        lax.fori_loop(0, CHG, scale, 0)
        pltpu.async_copy(bR, acc_sh.at[dsc], ssem, add=True)

        @pl.when(i + 2 < nchunk)
        def _():
            issue_idx(i + 2, s)

    def body(i, _):
        @pl.when(i % 2 == 0)
        def _():
            process(i, 0)

        @pl.when(i % 2 == 1)
        def _():
            process(i, 1)
        return 0

    lax.fori_loop(0, nchunk, body, 0)
    drain_scat(0)
    drain_scat(1)
    plsc.subcore_barrier()

    for q in range(_STRIPE // CHG):
        pltpu.sync_copy(acc_sh.at[pl.ds(s0 + q * CHG, CHG)],
                        accp_hbm.at[cid, pl.ds(s0 + q * CHG, CHG)])


def _gcn_scatter(hp, dinv_flat, srcp, dstp, wp):
    return pl.kernel(
        _gcn_body,
        out_type=jax.ShapeDtypeStruct((NC, N_PAD, H), _f32),
        mesh=_mesh,
        compiler_params=pltpu.CompilerParams(needs_layout_passes=False),
        scratch_types=[
            pltpu.VMEM((CHG,), _i32),
            pltpu.VMEM((CHG,), _i32),
            pltpu.VMEM((CHG,), _i32),
            pltpu.VMEM((CHG,), _i32),
            pltpu.VMEM((CHG,), _i32),
            pltpu.VMEM((CHG,), _i32),
            pltpu.VMEM((CHG,), _f32),
            pltpu.VMEM((CHG,), _f32),
            pltpu.VMEM((CHG, H), _f32),
            pltpu.VMEM((CHG, H), _f32),
            pltpu.VMEM((NP_,), _f32),
            pltpu.VMEM((CHG,), _f32),
            pltpu.VMEM_SHARED((N_PAD, H), _f32),
            pltpu.SemaphoreType.DMA,
            pltpu.SemaphoreType.DMA,
            pltpu.SemaphoreType.DMA,
            pltpu.SemaphoreType.DMA,
            pltpu.SemaphoreType.DMA,
            pltpu.SemaphoreType.DMA,
        ],
    )(hp, dinv_flat, srcp, dstp, wp)


def _gcn0_fin_body(accp_ref, h0_ref, dinv_ref, gb0_ref, gW1_ref, h1_ref):
    di = dinv_ref[...]
    acc = accp_ref[0] + accp_ref[1]
    x2 = jax.nn.relu(di * acc + di * di * h0_ref[...] + gb0_ref[...])
    h1_ref[...] = jnp.dot(x2, gW1_ref[...], preferred_element_type=_f32)


def _gcn0_finish(accp, h0, dinv, gb0, gW1):
    nb = N // NBLK
    full = lambda s: pl.BlockSpec(s, lambda i: tuple(0 for _ in s))
    return pl.pallas_call(
        _gcn0_fin_body,
        grid=(nb,),
        in_specs=[pl.BlockSpec((NC, NBLK, H), lambda i: (0, i, 0)),
                  pl.BlockSpec((NBLK, H), lambda i: (i, 0)),
                  pl.BlockSpec((NBLK, 1), lambda i: (i, 0)),
                  full((1, H)), full((H, H))],
        out_specs=pl.BlockSpec((NBLK, H), lambda i: (i, 0)),
        out_shape=jax.ShapeDtypeStruct((N, H), _f32),
    )(accp, h0, dinv, gb0, gW1)


def _gru_body(accp_ref, h1_ref, dinv_ref, gb1_ref, gh_ref, ps_ref,
              Wih_ref, bih_ref, hnew_ref):
    di = dinv_ref[...]
    acc = accp_ref[0] + accp_ref[1]
    x3 = di * acc + di * di * h1_ref[...] + gb1_ref[...]
    gi = jnp.dot(x3, Wih_ref[...], preferred_element_type=_f32) + bih_ref[...]
    gh = gh_ref[...]
    r = jax.nn.sigmoid(gi[:, 0:H] + gh[:, 0:H])
    zg = jax.nn.sigmoid(gi[:, H:2 * H] + gh[:, H:2 * H])
    n = jnp.tanh(gi[:, 2 * H:3 * H] + r * gh[:, 2 * H:3 * H])
    hnew_ref[...] = (1.0 - zg) * n + zg * ps_ref[...]


def _gcn1_gru(accp, h1, dinv, gb1, gh, ps, W_ih, b_ih):
    nb = N // NBLK
    full = lambda s: pl.BlockSpec(s, lambda i: tuple(0 for _ in s))
    return pl.pallas_call(
        _gru_body,
        grid=(nb,),
        in_specs=[pl.BlockSpec((NC, NBLK, H), lambda i: (0, i, 0)),
                  pl.BlockSpec((NBLK, H), lambda i: (i, 0)),
                  pl.BlockSpec((NBLK, 1), lambda i: (i, 0)),
                  full((1, H)),
                  pl.BlockSpec((NBLK, 3 * H), lambda i: (i, 0)),
                  pl.BlockSpec((NBLK, H), lambda i: (i, 0)),
                  full((H, 3 * H)), full((1, 3 * H))],
        out_specs=pl.BlockSpec((NBLK, H), lambda i: (i, 0)),
        out_shape=jax.ShapeDtypeStruct((N, H), _f32),
    )(accp, h1, dinv, gb1, gh, ps, W_ih, b_ih)


_LPAD = 20480
_LPT = _LPAD // NW
_LCH = _LPT // CH


def _dec_gather_body(hn_hbm, lidx3_hbm, out_hbm, idxb, buf0, buf1,
                     sem0, sem1, wsem0, wsem1):
    wid = lax.axis_index("s") * NC + lax.axis_index("c")
    pltpu.sync_copy(lidx3_hbm.at[wid], idxb)

    slots = ((buf0, sem0, wsem0), (buf1, sem1, wsem1))

    def issue(ch, s):
        b, sm, _ = slots[s]
        pltpu.async_copy(hn_hbm.at[idxb.at[ch]], b, sm)

    issue(0, 0)

    def process(i, s):
        b, sm, ws = slots[s]
        bn, _, wsn = slots[1 - s]

        @pl.when(i + 1 < _LCH)
        def _():
            @pl.when(i >= 1)
            def _():
                pltpu.make_async_copy(bn, out_hbm.at[pl.ds(0, CH)], wsn).wait()
            issue(i + 1, 1 - s)

        pltpu.make_async_copy(hn_hbm.at[pl.ds(0, CH)], b, sm).wait()
        pltpu.async_copy(b, out_hbm.at[pl.ds(wid * _LPT + i * CH, CH)], ws)

    def body(i, _):
        @pl.when(i % 2 == 0)
        def _():
            process(i, 0)

        @pl.when(i % 2 == 1)
        def _():
            process(i, 1)
        return 0

    lax.fori_loop(0, _LCH, body, 0)
    pltpu.make_async_copy(buf0, out_hbm.at[pl.ds(0, CH)], wsem0).wait()
    pltpu.make_async_copy(buf1, out_hbm.at[pl.ds(0, CH)], wsem1).wait()


def _dec_gather(h_new, lidx3):
    return pl.kernel(
        _dec_gather_body,
        out_type=jax.ShapeDtypeStruct((_LPAD, H), _f32),
        mesh=_mesh,
        compiler_params=pltpu.CompilerParams(needs_layout_passes=False),
        scratch_types=[
            pltpu.VMEM((_LCH, CH), _i32),
            pltpu.VMEM((CH, H), _f32),
            pltpu.VMEM((CH, H), _f32),
            pltpu.SemaphoreType.DMA,
            pltpu.SemaphoreType.DMA,
            pltpu.SemaphoreType.DMA,
            pltpu.SemaphoreType.DMA,
        ],
    )(h_new, lidx3)


def _dec_body(Hs_ref, Hd_ref, dW1a_ref, dW1b_ref, db1_ref, dW2_ref, db2_ref,
              pred_ref):
    hid = jax.nn.relu(
        jnp.dot(Hs_ref[...], dW1a_ref[...], preferred_element_type=_f32)
        + jnp.dot(Hd_ref[...], dW1b_ref[...], preferred_element_type=_f32)
        + db1_ref[...])
    pred_ref[...] = jnp.dot(hid, dW2_ref[...], preferred_element_type=_f32) \
        + db2_ref[0, 0]


def _decoder(Hs, Hd, dW1a, dW1b, db1, dW2, db2):
    nb = LBL // NBLK
    full = lambda s: pl.BlockSpec(s, lambda i: tuple(0 for _ in s))
    return pl.pallas_call(
        _dec_body,
        grid=(nb,),
        in_specs=[pl.BlockSpec((NBLK, H), lambda i: (i, 0)),
                  pl.BlockSpec((NBLK, H), lambda i: (i, 0)),
                  full((H, H)), full((H, H)), full((1, H)), full((H, 1)),
                  full((1, 1))],
        out_specs=pl.BlockSpec((NBLK, 1), lambda i: (i, 0)),
        out_shape=jax.ShapeDtypeStruct((LBL, 1), _f32),
    )(Hs, Hd, dW1a, dW1b, db1, dW2, db2)


def kernel(x, edge_index, edge_label_index, edge_feature, previous_state,
           W1, b1, time_w, time_b, sW1, sb1, sW2, sb2, gW0, gb0, gW1, gb1,
           W_ih, W_hh, b_ih, b_hh, dW1, db1, dW2, db2):
    src = edge_index[0]
    dst = edge_index[1]
    sW1a = sW1[0:H]
    sW1b = sW1[H:2 * H]
    sW1c = sW1[2 * H:3 * H]
    b1r = b1.reshape(1, H)
    sb1r = sb1.reshape(1, H)
    sb2r = sb2.reshape(1, 1)
    tbr = time_b.reshape(1, H)
    gb0r = gb0.reshape(1, H)
    gb1r = gb1.reshape(1, H)
    bihr = b_ih.reshape(1, 3 * H)
    bhhr = b_hh.reshape(1, 3 * H)
    db1r = db1.reshape(1, H)
    db2r = db2.reshape(1, 1)

    src3 = src.reshape(NW, NCHUNK, CH)
    dst3 = dst.reshape(NW, NCHUNK, CH)

    x1, A, B, h0, gh = _dense_pre(x, previous_state, W1, b1r, sW1a, sW1b,
                                  gW0, W_hh, bhhr)

    G = _gather_add(A, B, src3, dst3)

    m = _min_reduce(edge_feature.reshape(E // H, H))

    wv = time_w[0]
    bv = time_b
    theta = 1.5 * wv + bv
    dd = jnp.arange(ND, dtype=_f32)
    ratio = jnp.where(dd[:, None] > 0, wv[None, :] / jnp.maximum(dd[:, None], 1.0),
                      jnp.ones((1, H), _f32))
    M = jnp.cumprod(ratio, axis=0)
    kk = jnp.floor_divide(jnp.arange(ND), 2).astype(_f32)
    sgn = jnp.where(jnp.mod(kk, 2.0) == 0.0, 1.0, -1.0)
    even = jnp.mod(jnp.arange(ND), 2) == 0
    gam = jnp.where(even[:, None], sgn[:, None] * jnp.cos(theta)[None, :],
                    -sgn[:, None] * jnp.sin(theta)[None, :])
    R = (M * gam) @ sW1c

    lg = _logits(G, edge_feature.reshape(E // EB, 1, EB), m, R, sb1r,
                 sW2, sb2r)

    w2 = _threshold(lg.reshape(E // H, H))
    w = w2.reshape(E)

    degp = _deg_scatter(dst, w)
    dinv = _dinv_kernel(degp.reshape(NW, NP_))
    dinv_flat = dinv.reshape(NP_)

    pad = jnp.zeros((E_PAD - E,), _i32)
    srcp = jnp.concatenate([src, pad])
    dstp = jnp.concatenate([dst, pad])
    wp = jnp.concatenate([w, jnp.zeros((E_PAD - E,), _f32)])
    accp0 = _gcn_scatter(h0, dinv_flat, srcp, dstp, wp)
    h1 = _gcn0_finish(accp0, h0, dinv, gb0r, gW1)

    accp1 = _gcn_scatter(h1, dinv_flat, srcp, dstp, wp)
    h_new = _gcn1_gru(accp1, h1, dinv, gb1r, gh, previous_state, W_ih, bihr)

    lidx = jnp.concatenate([edge_label_index[0], edge_label_index[1]])
    lidx_pad = jnp.concatenate(
        [lidx, jnp.zeros((_LPAD - 2 * LBL,), _i32)])
    HG = _dec_gather(h_new, lidx_pad.reshape(NW, _LCH, CH))
    Hs = HG[0:LBL]
    Hd = HG[LBL:2 * LBL]

    dW1a = dW1[0:H]
    dW1b = dW1[H:2 * H]
    pred = _decoder(Hs, Hd, dW1a, dW1b, db1r, dW2, db2r)
    return (pred, h_new)
